# Initial kernel scaffold; baseline (speedup 1.0000x reference)
import jax
import jax.numpy as jnp
from jax.experimental import pallas as pl


def kernel(x):
    raise NotImplementedError("write your pallas kernel here")


# speedup vs baseline: 1.4408x; 1.4408x over previous
import jax
import jax.numpy as jnp
from jax import lax
from jax.experimental import pallas as pl
from jax.experimental.pallas import tpu as pltpu
from jax.experimental.pallas import tpu_sc as plsc

N, C, H, W = 32, 768, 32, 32
HW = H * W
R = N * C
K = 205
ALPHA = 0.6
NC, NS = 2, 16
NW = NC * NS
RPW = R // NW
GPW = RPW // 16
GELEMS = 16 * HW

_PFXMASK = {16: 0xFF000000, 8: 0xFFFF0000, 0: 0xFFFFFF00}


def _u2f(u):
    bits = jnp.where(u >= jnp.uint32(0x80000000),
                     u ^ jnp.uint32(0x80000000), ~u)
    return plsc.bitcast(bits, jnp.float32)


def _body(x_hbm, out_hbm, xb, ub, ha, hb, ob):
    wid = lax.axis_index("s") * NC + lax.axis_index("c")
    iota = lax.iota(jnp.int32, 16)
    ones_i = jnp.ones((16,), jnp.int32)
    zero_i = jnp.zeros((16,), jnp.int32)
    zero_f = jnp.zeros((16,), jnp.float32)
    zero_u = jnp.zeros((16,), jnp.uint32)
    kvec = jnp.full((16,), K, jnp.int32)

    def clr(i, c):
        ha[pl.ds(i * 16, 16)] = zero_i
        hb[pl.ds(i * 16, 16)] = zero_i
        return c
    lax.fori_loop(0, 256, clr, 0)

    def scan_hist(h_ref, rem, desc, clear):
        def sbody(i, carry):
            cum, sel, base = carry
            b = (255 - i) if desc else i
            h = h_ref[pl.ds(b * 16, 16)]
            if clear:
                h_ref[pl.ds(b * 16, 16)] = zero_i
            ncum = cum + h
            take = (cum < rem) & (ncum >= rem)
            sel = jnp.where(take, b.astype(jnp.uint32), sel)
            base = jnp.where(take, cum, base)
            return ncum, sel, base
        _, sel, base = lax.fori_loop(0, 256, sbody, (zero_i, zero_u, zero_i))
        return sel, base

    def group(g, carry):
        off = (wid * GPW + g) * GELEMS
        pltpu.sync_copy(x_hbm.at[pl.ds(off, GELEMS)], xb)

        def p1(j, c):
            xv = plsc.load_gather(xb, [iota * HW + j])
            t = plsc.bitcast(xv, jnp.int32)
            s = t ^ ((t >> 31) & jnp.int32(0x7FFFFFFF))
            u = plsc.bitcast(s, jnp.uint32) ^ jnp.uint32(0x80000000)
            ub[pl.ds(j * 16, 16)] = u
            d = (u >> 24).astype(jnp.int32)
            plsc.addupdate_scatter(ha, [d * 16 + iota], ones_i)
            return c
        lax.fori_loop(0, HW, p1, 0)

        sel, base = scan_hist(ha, kvec, desc=True, clear=False)
        pfx_hi = sel << 24
        rem_hi = kvec - base
        sel, base = scan_hist(ha, kvec, desc=False, clear=True)
        pfx_lo = sel << 24
        rem_lo = kvec - base

        for shift in (16, 8, 0):
            maskc = jnp.uint32(_PFXMASK[shift])
            def refine(j, c, maskc=maskc, shift=shift,
                       pfx_hi=pfx_hi, pfx_lo=pfx_lo):
                u = ub[pl.ds(j * 16, 16)]
                w = u & maskc
                d = ((u >> shift) & jnp.uint32(0xFF)).astype(jnp.int32)
                idx = d * 16 + iota
                plsc.addupdate_scatter(ha, [idx], ones_i, mask=(w == pfx_hi))
                plsc.addupdate_scatter(hb, [idx], ones_i, mask=(w == pfx_lo))
                return c
            lax.fori_loop(0, HW, refine, 0)
            sel, base = scan_hist(ha, rem_hi, desc=True, clear=True)
            pfx_hi = pfx_hi | (sel << shift)
            rem_hi = rem_hi - base
            sel, base = scan_hist(hb, rem_lo, desc=False, clear=True)
            pfx_lo = pfx_lo | (sel << shift)
            rem_lo = rem_lo - base

        thi, tlo = pfx_hi, pfx_lo

        def fin(j, carry):
            sgt, cgt, slt, clt = carry
            u = ub[pl.ds(j * 16, 16)]
            xv = _u2f(u)
            gt = u > thi
            lt = u < tlo
            sgt = sgt + jnp.where(gt, xv, zero_f)
            cgt = cgt + jnp.where(gt, ones_i, zero_i)
            slt = slt + jnp.where(lt, xv, zero_f)
            clt = clt + jnp.where(lt, ones_i, zero_i)
            return sgt, cgt, slt, clt
        sgt, cgt, slt, clt = lax.fori_loop(
            0, HW, fin, (zero_f, zero_i, zero_f, zero_i))

        topsum = sgt + (kvec - cgt).astype(jnp.float32) * _u2f(thi)
        botsum = slt + (kvec - clt).astype(jnp.float32) * _u2f(tlo)
        res = topsum * jnp.float32(1.0 / K) + botsum * jnp.float32(ALPHA / K)
        ob[pl.ds(g * 16, 16)] = res
        return carry

    lax.fori_loop(0, GPW, group, 0)
    pltpu.sync_copy(ob, out_hbm.at[pl.ds(wid * RPW, RPW)])


def _make(interpret=False):
    return pl.kernel(
        _body,
        out_type=jax.ShapeDtypeStruct((R,), jnp.float32),
        mesh=plsc.VectorSubcoreMesh(core_axis_name="c", subcore_axis_name="s",
                                    num_cores=NC, num_subcores=NS),
        scratch_types=[
            pltpu.VMEM((GELEMS,), jnp.float32),
            pltpu.VMEM((GELEMS,), jnp.uint32),
            pltpu.VMEM((256 * 16,), jnp.int32),
            pltpu.VMEM((256 * 16,), jnp.int32),
            pltpu.VMEM((RPW,), jnp.float32),
        ],
        compiler_params=pltpu.CompilerParams(needs_layout_passes=False),
        interpret=interpret,
    )


_wildcat = _make()


@jax.jit
def kernel(x):
    out = _wildcat(x.reshape(R * HW))
    return out.reshape(N, C)


# speedup vs baseline: 1.6027x; 1.1124x over previous
import jax
import jax.numpy as jnp
from jax import lax
from jax.experimental import pallas as pl
from jax.experimental.pallas import tpu as pltpu
from jax.experimental.pallas import tpu_sc as plsc

N, C, H, W = 32, 768, 32, 32
HW = H * W
R = N * C
K = 205
ALPHA = 0.6
NC, NS = 2, 16
NW = NC * NS
RPW = R // NW
GPW = RPW // 16
GELEMS = 16 * HW

_PFXMASK = {16: 0xFF000000, 8: 0xFFFF0000, 0: 0xFFFFFF00}


def _u2f(u):
    bits = jnp.where(u >= jnp.uint32(0x80000000),
                     u ^ jnp.uint32(0x80000000), ~u)
    return plsc.bitcast(bits, jnp.float32)


_U = 8
_SU = 8


def _body(x_hbm, out_hbm, xb, ub, ha, hb, ob):
    wid = lax.axis_index("s") * NC + lax.axis_index("c")
    iota = lax.iota(jnp.int32, 16)
    iota_hw = iota * HW
    ones_i = jnp.ones((16,), jnp.int32)
    zero_i = jnp.zeros((16,), jnp.int32)
    zero_f = jnp.zeros((16,), jnp.float32)
    zero_u = jnp.zeros((16,), jnp.uint32)
    kvec = jnp.full((16,), K, jnp.int32)

    def clr(i, c):
        ha[pl.ds(i * 16, 16)] = zero_i
        hb[pl.ds(i * 16, 16)] = zero_i
        return c
    lax.fori_loop(0, 256, clr, 0)

    def scan_hist(h_ref, rem, desc, clear):
        def sbody(i, carry):
            cum, sel, base = carry
            for t in range(_SU):
                b = (255 - (i * _SU + t)) if desc else (i * _SU + t)
                h = h_ref[pl.ds(b * 16, 16)]
                if clear:
                    h_ref[pl.ds(b * 16, 16)] = zero_i
                ncum = cum + h
                take = (cum < rem) & (ncum >= rem)
                sel = jnp.where(take, b.astype(jnp.uint32), sel)
                base = jnp.where(take, cum, base)
                cum = ncum
            return cum, sel, base
        _, sel, base = lax.fori_loop(0, 256 // _SU, sbody,
                                     (zero_i, zero_u, zero_i))
        return sel, base

    def group(g, carry):
        off = (wid * GPW + g) * GELEMS
        pltpu.sync_copy(x_hbm.at[pl.ds(off, GELEMS)], xb)

        def p1(i, c):
            for t in range(_U):
                j = i * _U + t
                xv = plsc.load_gather(xb, [iota_hw + j])
                tt = plsc.bitcast(xv, jnp.int32)
                s = tt ^ ((tt >> 31) & jnp.int32(0x7FFFFFFF))
                u = plsc.bitcast(s, jnp.uint32) ^ jnp.uint32(0x80000000)
                ub[pl.ds(j * 16, 16)] = u
                d = (u >> 24).astype(jnp.int32)
                plsc.addupdate_scatter(ha, [d * 16 + iota], ones_i)
            return c
        lax.fori_loop(0, HW // _U, p1, 0)

        sel, base = scan_hist(ha, kvec, desc=True, clear=False)
        pfx_hi = sel << 24
        rem_hi = kvec - base
        sel, base = scan_hist(ha, kvec, desc=False, clear=True)
        pfx_lo = sel << 24
        rem_lo = kvec - base

        for shift in (16, 8, 0):
            maskc = jnp.uint32(_PFXMASK[shift])
            def refine(i, c, maskc=maskc, shift=shift,
                       pfx_hi=pfx_hi, pfx_lo=pfx_lo):
                for t in range(_U):
                    j = i * _U + t
                    u = ub[pl.ds(j * 16, 16)]
                    w = u & maskc
                    d = ((u >> shift) & jnp.uint32(0xFF)).astype(jnp.int32)
                    idx = d * 16 + iota
                    plsc.addupdate_scatter(ha, [idx], ones_i,
                                           mask=(w == pfx_hi))
                    plsc.addupdate_scatter(hb, [idx], ones_i,
                                           mask=(w == pfx_lo))
                return c
            lax.fori_loop(0, HW // _U, refine, 0)
            sel, base = scan_hist(ha, rem_hi, desc=True, clear=True)
            pfx_hi = pfx_hi | (sel << shift)
            rem_hi = rem_hi - base
            sel, base = scan_hist(hb, rem_lo, desc=False, clear=True)
            pfx_lo = pfx_lo | (sel << shift)
            rem_lo = rem_lo - base

        thi, tlo = pfx_hi, pfx_lo

        def fin(i, carry):
            sgt, cgt, slt, clt = carry
            for t in range(_U):
                j = i * _U + t
                u = ub[pl.ds(j * 16, 16)]
                xv = _u2f(u)
                gt = u > thi
                lt = u < tlo
                sgt = sgt + jnp.where(gt, xv, zero_f)
                cgt = cgt + jnp.where(gt, ones_i, zero_i)
                slt = slt + jnp.where(lt, xv, zero_f)
                clt = clt + jnp.where(lt, ones_i, zero_i)
            return sgt, cgt, slt, clt
        sgt, cgt, slt, clt = lax.fori_loop(
            0, HW // _U, fin, (zero_f, zero_i, zero_f, zero_i))

        topsum = sgt + (kvec - cgt).astype(jnp.float32) * _u2f(thi)
        botsum = slt + (kvec - clt).astype(jnp.float32) * _u2f(tlo)
        res = topsum * jnp.float32(1.0 / K) + botsum * jnp.float32(ALPHA / K)
        ob[pl.ds(g * 16, 16)] = res
        return carry

    lax.fori_loop(0, GPW, group, 0)
    pltpu.sync_copy(ob, out_hbm.at[pl.ds(wid * RPW, RPW)])


def _make(interpret=False):
    return pl.kernel(
        _body,
        out_type=jax.ShapeDtypeStruct((R,), jnp.float32),
        mesh=plsc.VectorSubcoreMesh(core_axis_name="c", subcore_axis_name="s",
                                    num_cores=NC, num_subcores=NS),
        scratch_types=[
            pltpu.VMEM((GELEMS,), jnp.float32),
            pltpu.VMEM((GELEMS,), jnp.uint32),
            pltpu.VMEM((256 * 16,), jnp.int32),
            pltpu.VMEM((256 * 16,), jnp.int32),
            pltpu.VMEM((RPW,), jnp.float32),
        ],
        compiler_params=pltpu.CompilerParams(needs_layout_passes=False),
        interpret=interpret,
    )


_wildcat = _make()


@jax.jit
def kernel(x):
    out = _wildcat(x.reshape(R * HW))
    return out.reshape(N, C)


# speedup vs baseline: 4.4323x; 2.7656x over previous
import jax
import jax.numpy as jnp
from jax import lax
from jax.experimental import pallas as pl
from jax.experimental.pallas import tpu as pltpu
from jax.experimental.pallas import tpu_sc as plsc

N, C, H, W = 32, 768, 32, 32
HW = H * W
R = N * C
K = 205
ALPHA = 0.6
NC, NS = 2, 16
NW = NC * NS
RPW = R // NW
GPW = RPW // 16
GELEMS = 16 * HW

_PFXMASK = {16: 0xFF000000, 8: 0xFFFF0000, 0: 0xFFFFFF00}

_U = 8
_SU = 8


def _u2f(u):
    bits = jnp.where(u >= jnp.uint32(0x80000000),
                     u ^ jnp.uint32(0x80000000), ~u)
    return plsc.bitcast(bits, jnp.float32)


def _treesum(vals):
    while len(vals) > 1:
        vals = [a + b for a, b in zip(vals[::2], vals[1::2])] + (
            [vals[-1]] if len(vals) % 2 else [])
    return vals[0]


def _body(x_hbm, out_hbm, xb0, xb1, ub, ha, hb, ob, sem0, sem1):
    wid = lax.axis_index("s") * NC + lax.axis_index("c")
    iota = lax.iota(jnp.int32, 16)
    iota_hw = iota * HW
    ones_i = jnp.ones((16,), jnp.int32)
    zero_i = jnp.zeros((16,), jnp.int32)
    zero_f = jnp.zeros((16,), jnp.float32)
    zero_u = jnp.zeros((16,), jnp.uint32)
    kvec = jnp.full((16,), K, jnp.int32)

    def clr(i, c):
        ha[pl.ds(i * 16, 16)] = zero_i
        hb[pl.ds(i * 16, 16)] = zero_i
        return c
    lax.fori_loop(0, 256, clr, 0)

    def off(g):
        return (wid * GPW + g) * GELEMS

    def copy(g, xb, sem):
        return pltpu.make_async_copy(x_hbm.at[pl.ds(off(g), GELEMS)], xb, sem)

    def scan_hist(h_ref, rem, desc, clear):
        def sbody(i, carry):
            cum, sel, base = carry
            bs = [(255 - (i * _SU + t)) if desc else (i * _SU + t)
                  for t in range(_SU)]
            hs = [h_ref[pl.ds(b * 16, 16)] for b in bs]
            if clear:
                for b in bs:
                    h_ref[pl.ds(b * 16, 16)] = zero_i
            for b, h in zip(bs, hs):
                ncum = cum + h
                take = (cum < rem) & (ncum >= rem)
                sel = jnp.where(take, b.astype(jnp.uint32), sel)
                base = jnp.where(take, cum, base)
                cum = ncum
            return cum, sel, base
        _, sel, base = lax.fori_loop(0, 256 // _SU, sbody,
                                     (zero_i, zero_u, zero_i))
        return sel, base

    def work(g, xb):
        def skew_idx(j):
            return iota_hw + ((j + iota) & jnp.int32(HW - 1))

        def p1(i, c):
            js = [i * _U + t for t in range(_U)]
            xs = [plsc.load_gather(xb, [skew_idx(j)]) for j in js]
            ts = [plsc.bitcast(x, jnp.int32) for x in xs]
            ms = [(t >> 31) & jnp.int32(0x7FFFFFFF) for t in ts]
            ss = [t ^ m for t, m in zip(ts, ms)]
            us = [plsc.bitcast(s, jnp.uint32) ^ jnp.uint32(0x80000000)
                  for s in ss]
            ds = [(u >> 24).astype(jnp.int32) for u in us]
            idxs = [d * 16 + iota for d in ds]
            for j, u in zip(js, us):
                ub[pl.ds(j * 16, 16)] = u
            for idx in idxs:
                plsc.addupdate_scatter(ha, [idx], ones_i)
            return c
        lax.fori_loop(0, HW // _U, p1, 0)

        sel, base = scan_hist(ha, kvec, desc=True, clear=False)
        pfx_hi = sel << 24
        rem_hi = kvec - base
        sel, base = scan_hist(ha, kvec, desc=False, clear=True)
        pfx_lo = sel << 24
        rem_lo = kvec - base

        for shift in (16, 8, 0):
            maskc = jnp.uint32(_PFXMASK[shift])
            def refine(i, c, maskc=maskc, shift=shift,
                       pfx_hi=pfx_hi, pfx_lo=pfx_lo):
                js = [i * _U + t for t in range(_U)]
                us = [ub[pl.ds(j * 16, 16)] for j in js]
                ws = [u & maskc for u in us]
                mhi = [w == pfx_hi for w in ws]
                mlo = [w == pfx_lo for w in ws]
                if shift:
                    ds = [((u >> shift) & jnp.uint32(0xFF)).astype(jnp.int32)
                          for u in us]
                else:
                    ds = [(u & jnp.uint32(0xFF)).astype(jnp.int32) for u in us]
                idxs = [d * 16 + iota for d in ds]
                for idx, mh, ml in zip(idxs, mhi, mlo):
                    plsc.addupdate_scatter(ha, [idx], ones_i, mask=mh)
                    plsc.addupdate_scatter(hb, [idx], ones_i, mask=ml)
                return c
            lax.fori_loop(0, HW // _U, refine, 0)
            sel, base = scan_hist(ha, rem_hi, desc=True, clear=True)
            pfx_hi = pfx_hi | (sel << shift)
            rem_hi = rem_hi - base
            sel, base = scan_hist(hb, rem_lo, desc=False, clear=True)
            pfx_lo = pfx_lo | (sel << shift)
            rem_lo = rem_lo - base

        thi, tlo = pfx_hi, pfx_lo

        def fin(i, carry):
            sgt, slt = carry
            js = [i * _U + t for t in range(_U)]
            us = [ub[pl.ds(j * 16, 16)] for j in js]
            xs = [_u2f(u) for u in us]
            gts = [u > thi for u in us]
            lts = [u < tlo for u in us]
            cg = [jnp.where(m, x, zero_f) for m, x in zip(gts, xs)]
            cl = [jnp.where(m, x, zero_f) for m, x in zip(lts, xs)]
            return sgt + _treesum(cg), slt + _treesum(cl)
        sgt, slt = lax.fori_loop(0, HW // _U, fin, (zero_f, zero_f))

        topsum = sgt + rem_hi.astype(jnp.float32) * _u2f(thi)
        botsum = slt + rem_lo.astype(jnp.float32) * _u2f(tlo)
        res = topsum * jnp.float32(1.0 / K) + botsum * jnp.float32(ALPHA / K)
        ob[pl.ds(g * 16, 16)] = res

    copy(0, xb0, sem0).start()
    copy(1, xb1, sem1).start()

    def pair(i, c):
        g0 = i * 2
        copy(g0, xb0, sem0).wait()
        work(g0, xb0)

        @pl.when(i < GPW // 2 - 1)
        def _():
            copy(g0 + 2, xb0, sem0).start()

        g1 = g0 + 1
        copy(g1, xb1, sem1).wait()
        work(g1, xb1)

        @pl.when(i < GPW // 2 - 1)
        def _():
            copy(g1 + 2, xb1, sem1).start()
        return c

    lax.fori_loop(0, GPW // 2, pair, 0)
    pltpu.sync_copy(ob, out_hbm.at[pl.ds(wid * RPW, RPW)])


def _make(interpret=False):
    return pl.kernel(
        _body,
        out_type=jax.ShapeDtypeStruct((R,), jnp.float32),
        mesh=plsc.VectorSubcoreMesh(core_axis_name="c", subcore_axis_name="s",
                                    num_cores=NC, num_subcores=NS),
        scratch_types=[
            pltpu.VMEM((GELEMS,), jnp.float32),
            pltpu.VMEM((GELEMS,), jnp.float32),
            pltpu.VMEM((GELEMS,), jnp.uint32),
            pltpu.VMEM((256 * 16,), jnp.int32),
            pltpu.VMEM((256 * 16,), jnp.int32),
            pltpu.VMEM((RPW,), jnp.float32),
            pltpu.SemaphoreType.DMA,
            pltpu.SemaphoreType.DMA,
        ],
        compiler_params=pltpu.CompilerParams(needs_layout_passes=False),
        interpret=interpret,
    )


_wildcat = _make()


@jax.jit
def kernel(x):
    out = _wildcat(x.reshape(R * HW))
    return out.reshape(N, C)


# speedup vs baseline: 4.8054x; 1.0842x over previous
import jax
import jax.numpy as jnp
from jax import lax
from jax.experimental import pallas as pl
from jax.experimental.pallas import tpu as pltpu
from jax.experimental.pallas import tpu_sc as plsc

N, C, H, W = 32, 768, 32, 32
HW = H * W
R = N * C
K = 205
ALPHA = 0.6
NC, NS = 2, 16
NW = NC * NS
RPW = R // NW
GPW = RPW // 16
GELEMS = 16 * HW

_PFXMASK = {16: 0xFF000000, 8: 0xFFFF0000, 0: 0xFFFFFF00}

_U = 8
_RU = 16
_SU = 8


def _u2f(u):
    bits = jnp.where(u >= jnp.uint32(0x80000000),
                     u ^ jnp.uint32(0x80000000), ~u)
    return plsc.bitcast(bits, jnp.float32)


def _treesum(vals):
    while len(vals) > 1:
        vals = [a + b for a, b in zip(vals[::2], vals[1::2])] + (
            [vals[-1]] if len(vals) % 2 else [])
    return vals[0]


def _body(x_hbm, out_hbm, xb0, xb1, ub, ha, hb, ob, sem0, sem1):
    wid = lax.axis_index("s") * NC + lax.axis_index("c")
    iota = lax.iota(jnp.int32, 16)
    iota_hw = iota * HW
    ones_i = jnp.ones((16,), jnp.int32)
    zero_i = jnp.zeros((16,), jnp.int32)
    zero_f = jnp.zeros((16,), jnp.float32)
    zero_u = jnp.zeros((16,), jnp.uint32)
    kvec = jnp.full((16,), K, jnp.int32)

    def clr(i, c):
        ha[pl.ds(i * 16, 16)] = zero_i
        hb[pl.ds(i * 16, 16)] = zero_i
        return c
    lax.fori_loop(0, 256, clr, 0)

    def off(g):
        return (wid * GPW + g) * GELEMS

    def copy(g, xb, sem):
        return pltpu.make_async_copy(x_hbm.at[pl.ds(off(g), GELEMS)], xb, sem)

    def scan_hist(h_ref, rem, desc, clear):
        def sbody(i, carry):
            cum, sel, base = carry
            bs = [(255 - (i * _SU + t)) if desc else (i * _SU + t)
                  for t in range(_SU)]
            hs = [h_ref[pl.ds(b * 16, 16)] for b in bs]
            if clear:
                for b in bs:
                    h_ref[pl.ds(b * 16, 16)] = zero_i
            for b, h in zip(bs, hs):
                ncum = cum + h
                take = (cum < rem) & (ncum >= rem)
                sel = jnp.where(take, b.astype(jnp.uint32), sel)
                base = jnp.where(take, cum, base)
                cum = ncum
            return cum, sel, base
        _, sel, base = lax.fori_loop(0, 256 // _SU, sbody,
                                     (zero_i, zero_u, zero_i))
        return sel, base

    skew_const = iota_hw ^ iota

    def work(g, xb):
        def skew_idx(j):
            return skew_const ^ j

        def p1(i, c):
            js = [i * _U + t for t in range(_U)]
            xs = [plsc.load_gather(xb, [skew_idx(j)]) for j in js]
            ts = [plsc.bitcast(x, jnp.int32) for x in xs]
            ms = [(t >> 31) & jnp.int32(0x7FFFFFFF) for t in ts]
            ss = [t ^ m for t, m in zip(ts, ms)]
            us = [plsc.bitcast(s, jnp.uint32) ^ jnp.uint32(0x80000000)
                  for s in ss]
            ds = [(u >> 24).astype(jnp.int32) for u in us]
            idxs = [d * 16 + iota for d in ds]
            for j, u in zip(js, us):
                ub[pl.ds(j * 16, 16)] = u
            for idx in idxs:
                plsc.addupdate_scatter(ha, [idx], ones_i)
            return c
        lax.fori_loop(0, HW // _U, p1, 0)

        sel, base = scan_hist(ha, kvec, desc=True, clear=False)
        pfx_hi = sel << 24
        rem_hi = kvec - base
        sel, base = scan_hist(ha, kvec, desc=False, clear=True)
        pfx_lo = sel << 24
        rem_lo = kvec - base

        for shift in (16, 8, 0):
            maskc = jnp.uint32(_PFXMASK[shift])
            def refine(i, c, maskc=maskc, shift=shift,
                       pfx_hi=pfx_hi, pfx_lo=pfx_lo):
                js = [i * _RU + t for t in range(_RU)]
                us = [ub[pl.ds(j * 16, 16)] for j in js]
                ws = [u & maskc for u in us]
                mhi = [w == pfx_hi for w in ws]
                mlo = [w == pfx_lo for w in ws]
                if shift:
                    ds = [((u >> shift) & jnp.uint32(0xFF)).astype(jnp.int32)
                          for u in us]
                else:
                    ds = [(u & jnp.uint32(0xFF)).astype(jnp.int32) for u in us]
                idxs = [d * 16 + iota for d in ds]
                for idx, mh, ml in zip(idxs, mhi, mlo):
                    plsc.addupdate_scatter(ha, [idx], ones_i, mask=mh)
                    plsc.addupdate_scatter(hb, [idx], ones_i, mask=ml)
                return c
            lax.fori_loop(0, HW // _RU, refine, 0)
            sel, base = scan_hist(ha, rem_hi, desc=True, clear=True)
            pfx_hi = pfx_hi | (sel << shift)
            rem_hi = rem_hi - base
            sel, base = scan_hist(hb, rem_lo, desc=False, clear=True)
            pfx_lo = pfx_lo | (sel << shift)
            rem_lo = rem_lo - base

        thi, tlo = pfx_hi, pfx_lo

        def fin(i, carry):
            sgt, slt = carry
            js = [i * _U + t for t in range(_U)]
            us = [ub[pl.ds(j * 16, 16)] for j in js]
            xs = [plsc.load_gather(xb, [skew_idx(j)]) for j in js]
            gts = [u > thi for u in us]
            lts = [u < tlo for u in us]
            cg = [jnp.where(m, x, zero_f) for m, x in zip(gts, xs)]
            cl = [jnp.where(m, x, zero_f) for m, x in zip(lts, xs)]
            return sgt + _treesum(cg), slt + _treesum(cl)
        sgt, slt = lax.fori_loop(0, HW // _U, fin, (zero_f, zero_f))

        topsum = sgt + rem_hi.astype(jnp.float32) * _u2f(thi)
        botsum = slt + rem_lo.astype(jnp.float32) * _u2f(tlo)
        res = topsum * jnp.float32(1.0 / K) + botsum * jnp.float32(ALPHA / K)
        ob[pl.ds(g * 16, 16)] = res

    copy(0, xb0, sem0).start()
    copy(1, xb1, sem1).start()

    def pair(i, c):
        g0 = i * 2
        copy(g0, xb0, sem0).wait()
        work(g0, xb0)

        @pl.when(i < GPW // 2 - 1)
        def _():
            copy(g0 + 2, xb0, sem0).start()

        g1 = g0 + 1
        copy(g1, xb1, sem1).wait()
        work(g1, xb1)

        @pl.when(i < GPW // 2 - 1)
        def _():
            copy(g1 + 2, xb1, sem1).start()
        return c

    lax.fori_loop(0, GPW // 2, pair, 0)
    pltpu.sync_copy(ob, out_hbm.at[pl.ds(wid * RPW, RPW)])


def _make(interpret=False):
    return pl.kernel(
        _body,
        out_type=jax.ShapeDtypeStruct((R,), jnp.float32),
        mesh=plsc.VectorSubcoreMesh(core_axis_name="c", subcore_axis_name="s",
                                    num_cores=NC, num_subcores=NS),
        scratch_types=[
            pltpu.VMEM((GELEMS,), jnp.float32),
            pltpu.VMEM((GELEMS,), jnp.float32),
            pltpu.VMEM((GELEMS,), jnp.uint32),
            pltpu.VMEM((256 * 16,), jnp.int32),
            pltpu.VMEM((256 * 16,), jnp.int32),
            pltpu.VMEM((RPW,), jnp.float32),
            pltpu.SemaphoreType.DMA,
            pltpu.SemaphoreType.DMA,
        ],
        compiler_params=pltpu.CompilerParams(needs_layout_passes=False),
        interpret=interpret,
    )


_wildcat = _make()


@jax.jit
def kernel(x):
    out = _wildcat(x.reshape(R * HW))
    return out.reshape(N, C)
